# baseline (speedup 1.0000x reference)
import functools

import jax
import jax.numpy as jnp
import numpy as np
from jax import lax
from jax.experimental import pallas as pl
from jax.experimental.pallas import tpu as pltpu
from jax.experimental.pallas import tpu_sc as plsc

N_TOKENS = 100000
D = 128
D4 = 32
B, L = 4096, 200
N = B * L

NC, NS, LANES = 2, 16, 16
NW = NC * NS
NPW = N // NW
K = 128
CHUNKS = NPW // K
RING = 4

_COL_PERM = tuple(
    32 * m + 16 * half + i
    for m in range(4) for i in range(16) for half in range(2)
)

_MASK_HI = np.int32(-65536)


_ROWS = 10000


def _tt_body(tbl_ref, w_ref, c_ref, out_ref):
    y = jnp.dot(tbl_ref[...], w_ref[0], preferred_element_type=jnp.float32)
    sel = jnp.where(pl.program_id(0) * _ROWS < N_TOKENS, 1.0, 0.0)
    out_ref[...] = (y + sel * c_ref[...]).astype(jnp.bfloat16)


def _make_tt(big_table, w3, c_row):
    grid = (3 * N_TOKENS) // _ROWS
    return pl.pallas_call(
        _tt_body,
        grid=(grid,),
        in_specs=[
            pl.BlockSpec((_ROWS, D4), lambda i: (i, 0)),
            pl.BlockSpec((1, D4, D), lambda i: ((i * _ROWS) // N_TOKENS, 0, 0)),
            pl.BlockSpec((1, D), lambda i: (0, 0)),
        ],
        out_specs=pl.BlockSpec((_ROWS, D), lambda i: (i, 0)),
        out_shape=jax.ShapeDtypeStruct((3 * N_TOKENS, D), jnp.bfloat16),
    )(big_table, w3, c_row)


def _uc_body(p_ref, w_ref, pb_ref, out_ref):
    out_ref[...] = jnp.dot(p_ref[...], w_ref[...],
                           preferred_element_type=jnp.float32) + pb_ref[...]


def _make_uc(p8, w3v, pb8):
    return pl.pallas_call(
        _uc_body,
        out_shape=jax.ShapeDtypeStruct((8, D), jnp.float32),
    )(p8, w3v, pb8)


def _sc_body(seqb_hbm, tt_hbm, uc_hbm, out_hbm,
             seq0, seq1, seq2, seq3,
             ti0, ti1, ti2, ti3, ai0, ai1, ai2, ai3, bi0, bi1, bi2, bi3,
             ac0, ac1, ac2, ac3, st0, st1, uc_v,
             gs0, gs1, gs2, gs3, os0, os1):
    seqv = [seq0, seq1, seq2, seq3]
    tiv = [ti0, ti1, ti2, ti3]
    aiv = [ai0, ai1, ai2, ai3]
    biv = [bi0, bi1, bi2, bi3]
    accv = [ac0, ac1, ac2, ac3]
    stv = [st0, st1]
    gsem = [gs0, gs1, gs2, gs3]
    osem = [os0, os1]

    wid = lax.axis_index("s") * NC + lax.axis_index("c")
    c_base = wid * CHUNKS

    pltpu.sync_copy(uc_hbm, uc_v)
    u_rows = [uc_v[0, pl.ds(16 * j, 16)] for j in range(D // 16)]

    def load_and_fire(ci, b):
        pltpu.sync_copy(seqb_hbm.at[c_base + ci], seqv[b])
        for g in range(K // LANES):
            s = pl.ds(g * LANES, LANES)
            tiv[b][s] = jnp.clip(seqv[b][0, s].astype(jnp.int32),
                                 0, N_TOKENS - 1)
            aiv[b][s] = jnp.clip(seqv[b][1, s].astype(jnp.int32),
                                 0, N_TOKENS - 1) + N_TOKENS
            biv[b][s] = jnp.clip(seqv[b][2, s].astype(jnp.int32),
                                 0, N_TOKENS - 1) + 2 * N_TOKENS

        def grp_body(g, carry):
            vblk = seqv[b][3, pl.ds(g * LANES, LANES)]
            for t in range(LANES):
                k = g * LANES + t
                vk = jnp.broadcast_to(vblk[t], (LANES,))
                for h in range(D // 32):
                    lo = plsc.bitcast(vk * u_rows[2 * h], jnp.int32)
                    hi = plsc.bitcast(vk * u_rows[2 * h + 1], jnp.int32)
                    w = (lax.shift_right_logical(lo, 16)
                         | (hi & _MASK_HI))
                    accv[b][k, pl.ds(32 * h, 32)] = plsc.bitcast(
                        w, jnp.bfloat16)
            return carry

        lax.fori_loop(0, K // LANES, grp_body, 0)
        pltpu.async_copy(tt_hbm.at[tiv[b]], accv[b], gsem[b], add=True)
        pltpu.async_copy(tt_hbm.at[aiv[b]], accv[b], gsem[b], add=True)
        pltpu.async_copy(tt_hbm.at[biv[b]], accv[b], gsem[b], add=True)

    def wait_gathers(b):
        pltpu.make_async_copy(tt_hbm.at[tiv[b]], accv[b], gsem[b]).wait()
        pltpu.make_async_copy(tt_hbm.at[aiv[b]], accv[b], gsem[b]).wait()
        pltpu.make_async_copy(tt_hbm.at[biv[b]], accv[b], gsem[b]).wait()

    def convert(b, sb):
        def grp_body(g, carry):
            for t in range(LANES):
                k = g * LANES + t
                for h in range(D // 32):
                    w = plsc.bitcast(accv[b][k, pl.ds(32 * h, 32)],
                                     jnp.int32)
                    stv[sb][k, pl.ds(32 * h, 16)] = plsc.bitcast(
                        w << 16, jnp.float32)
                    stv[sb][k, pl.ds(32 * h + 16, 16)] = plsc.bitcast(
                        w & _MASK_HI, jnp.float32)
            return carry

        lax.fori_loop(0, K // LANES, grp_body, 0)

    def out_slice(ci):
        return out_hbm.at[pl.ds((c_base + ci) * K, K)]

    def fire_out(ci, sb):
        pltpu.async_copy(stv[sb], out_slice(ci), osem[sb])

    def wait_out(ci, sb):
        pltpu.make_async_copy(stv[sb], out_slice(ci), osem[sb]).wait()

    for b in range(RING - 1):
        load_and_fire(b, b)

    def ring_body(p, carry):
        for q in range(RING):
            n = RING * p + q
            sq = q % 2
            m = n + RING - 1
            bm = (q + RING - 1) % RING

            @pl.when(m < CHUNKS)
            def _():
                load_and_fire(m, bm)

            wait_gathers(q)

            @pl.when(n >= 2)
            def _():
                wait_out(n - 2, sq)

            convert(q, sq)
            fire_out(n, sq)
        return carry

    lax.fori_loop(0, CHUNKS // RING, ring_body, 0)
    wait_out(CHUNKS - 2, 0)
    wait_out(CHUNKS - 1, 1)


@functools.cache
def _sc_gather_fn():
    return pl.kernel(
        _sc_body,
        out_type=jax.ShapeDtypeStruct((N, D), jnp.float32),
        mesh=plsc.VectorSubcoreMesh(core_axis_name="c", subcore_axis_name="s",
                                    num_cores=NC, num_subcores=NS),
        compiler_params=pltpu.CompilerParams(needs_layout_passes=False,
                                             use_tc_tiling_on_sc=False),
        scratch_types=(
            [pltpu.VMEM((4, K), jnp.float32) for _ in range(RING)]
            + [pltpu.VMEM((K,), jnp.int32) for _ in range(3 * RING)]
            + [pltpu.VMEM((K, D), jnp.bfloat16) for _ in range(RING)]
            + [pltpu.VMEM((K, D), jnp.float32) for _ in range(2)]
            + [pltpu.VMEM((8, D), jnp.float32)]
            + [pltpu.SemaphoreType.DMA for _ in range(RING + 2)]
        ),
    )


def kernel(seq, type_emb, node_a_emb, node_b_emb, vp_w, vp_b, proj_w, proj_b):
    p8 = jnp.zeros((8, D4), jnp.float32).at[0].set(vp_w[0]).at[1].set(vp_b)
    pb8 = jnp.zeros((8, D), jnp.float32).at[1].set(proj_b)
    uc = _make_uc(p8, proj_w[3 * D4:], pb8)

    big_table = jnp.concatenate([type_emb, node_a_emb, node_b_emb], axis=0)
    perm = jnp.asarray(_COL_PERM, dtype=jnp.int32)
    w3 = proj_w[: 3 * D4, perm].reshape(3, D4, D)
    c_perm = uc[1:2, perm]
    tt = _make_tt(big_table, w3, c_perm)

    seqb = seq.reshape(N // K, K, 4).transpose(0, 2, 1)
    out = _sc_gather_fn()(seqb, tt, uc)
    return out.reshape(B, L, D)


# speedup vs baseline: 1.7832x; 1.7832x over previous
import functools

import jax
import jax.numpy as jnp
from jax import lax
from jax.experimental import pallas as pl
from jax.experimental.pallas import tpu as pltpu
from jax.experimental.pallas import tpu_sc as plsc

N_TOKENS = 100000
D = 128
D4 = 32
B, L = 4096, 200
N = B * L

NC, NS, LANES = 2, 16, 16
NW = NC * NS
NPW = N // NW
K = 128
CHUNKS = NPW // K
RING = 5


_ROWS = 10000


def _tt_body(tbl_ref, w_ref, c_ref, out_ref):
    y = jnp.dot(tbl_ref[...], w_ref[0], preferred_element_type=jnp.float32)
    sel = jnp.where(pl.program_id(0) * _ROWS < N_TOKENS, 1.0, 0.0)
    out_ref[...] = y + sel * c_ref[...]


def _make_tt(big_table, w3, c_row):
    grid = (3 * N_TOKENS) // _ROWS
    return pl.pallas_call(
        _tt_body,
        grid=(grid,),
        in_specs=[
            pl.BlockSpec((_ROWS, D4), lambda i: (i, 0)),
            pl.BlockSpec((1, D4, D), lambda i: ((i * _ROWS) // N_TOKENS, 0, 0)),
            pl.BlockSpec((1, D), lambda i: (0, 0)),
        ],
        out_specs=pl.BlockSpec((_ROWS, D), lambda i: (i, 0)),
        out_shape=jax.ShapeDtypeStruct((3 * N_TOKENS, D), jnp.float32),
    )(big_table, w3, c_row)


def _uc_body(p_ref, w_ref, pb_ref, out_ref):
    out_ref[...] = jnp.dot(p_ref[...], w_ref[...],
                           preferred_element_type=jnp.float32) + pb_ref[...]


def _make_uc(p8, w3v, pb8):
    return pl.pallas_call(
        _uc_body,
        out_shape=jax.ShapeDtypeStruct((8, D), jnp.float32),
    )(p8, w3v, pb8)


def _sc_body(seqb_hbm, tt_hbm, uc_hbm, out_hbm, *scratch):
    it = iter(scratch)
    seqv = [next(it) for _ in range(RING)]
    tiv = [next(it) for _ in range(RING)]
    aiv = [next(it) for _ in range(RING)]
    biv = [next(it) for _ in range(RING)]
    rtv = [next(it) for _ in range(RING)]
    uc_v = next(it)
    gsem = [next(it) for _ in range(RING)]
    osem = [next(it) for _ in range(RING)]

    wid = lax.axis_index("s") * NC + lax.axis_index("c")
    c_base = wid * CHUNKS

    pltpu.sync_copy(uc_hbm, uc_v)
    u_rows = [uc_v[0, pl.ds(16 * j, 16)] for j in range(D // 16)]

    def load_and_fire(ci, b):
        pltpu.sync_copy(seqb_hbm.at[c_base + ci], seqv[b])
        for g in range(K // LANES):
            s = pl.ds(g * LANES, LANES)
            tiv[b][s] = jnp.clip(seqv[b][0, s].astype(jnp.int32),
                                 0, N_TOKENS - 1)
            aiv[b][s] = jnp.clip(seqv[b][1, s].astype(jnp.int32),
                                 0, N_TOKENS - 1) + N_TOKENS
            biv[b][s] = jnp.clip(seqv[b][2, s].astype(jnp.int32),
                                 0, N_TOKENS - 1) + 2 * N_TOKENS

        def grp_body(g, carry):
            vblk = seqv[b][3, pl.ds(g * LANES, LANES)]
            for t in range(LANES):
                k = g * LANES + t
                vk = jnp.broadcast_to(vblk[t], (LANES,))
                for j in range(D // 16):
                    rtv[b][k, pl.ds(16 * j, 16)] = vk * u_rows[j]
            return carry

        lax.fori_loop(0, K // LANES, grp_body, 0)
        pltpu.async_copy(tt_hbm.at[tiv[b]], rtv[b], gsem[b], add=True)
        pltpu.async_copy(tt_hbm.at[aiv[b]], rtv[b], gsem[b], add=True)
        pltpu.async_copy(tt_hbm.at[biv[b]], rtv[b], gsem[b], add=True)

    def wait_gathers(b):
        pltpu.make_async_copy(tt_hbm.at[tiv[b]], rtv[b], gsem[b]).wait()
        pltpu.make_async_copy(tt_hbm.at[aiv[b]], rtv[b], gsem[b]).wait()
        pltpu.make_async_copy(tt_hbm.at[biv[b]], rtv[b], gsem[b]).wait()

    def out_slice(ci):
        return out_hbm.at[pl.ds((c_base + ci) * K, K)]

    def fire_out(ci, b):
        pltpu.async_copy(rtv[b], out_slice(ci), osem[b])

    def wait_out(ci, b):
        pltpu.make_async_copy(rtv[b], out_slice(ci), osem[b]).wait()

    for b in range(RING - 1):
        load_and_fire(b, b)

    def ring_body(p, carry):
        for q in range(RING):
            n = RING * p + q
            wait_gathers(q)
            fire_out(n, q)
            m = n + RING - 1
            bm = (q + RING - 1) % RING

            @pl.when(m < CHUNKS)
            def _():
                @pl.when(m >= RING)
                def _():
                    wait_out(m - RING, bm)
                load_and_fire(m, bm)
        return carry

    lax.fori_loop(0, CHUNKS // RING, ring_body, 0)
    for q in range(RING):
        wait_out(CHUNKS - RING + q, q)


@functools.cache
def _sc_gather_fn():
    return pl.kernel(
        _sc_body,
        out_type=jax.ShapeDtypeStruct((N, D), jnp.float32),
        mesh=plsc.VectorSubcoreMesh(core_axis_name="c", subcore_axis_name="s",
                                    num_cores=NC, num_subcores=NS),
        compiler_params=pltpu.CompilerParams(needs_layout_passes=False),
        scratch_types=(
            [pltpu.VMEM((4, K), jnp.float32) for _ in range(RING)]
            + [pltpu.VMEM((K,), jnp.int32) for _ in range(3 * RING)]
            + [pltpu.VMEM((K, D), jnp.float32) for _ in range(RING)]
            + [pltpu.VMEM((8, D), jnp.float32)]
            + [pltpu.SemaphoreType.DMA for _ in range(2 * RING)]
        ),
    )


def kernel(seq, type_emb, node_a_emb, node_b_emb, vp_w, vp_b, proj_w, proj_b):
    p8 = jnp.zeros((8, D4), jnp.float32).at[0].set(vp_w[0]).at[1].set(vp_b)
    pb8 = jnp.zeros((8, D), jnp.float32).at[1].set(proj_b)
    uc = _make_uc(p8, proj_w[3 * D4:], pb8)

    big_table = jnp.concatenate([type_emb, node_a_emb, node_b_emb], axis=0)
    w3 = proj_w[: 3 * D4].reshape(3, D4, D)
    tt = _make_tt(big_table, w3, uc[1:2])

    seqb = seq.reshape(N // K, K, 4).transpose(0, 2, 1)
    out = _sc_gather_fn()(seqb, tt, uc)
    return out.reshape(B, L, D)


# speedup vs baseline: 2.0287x; 1.1377x over previous
import functools

import jax
import jax.numpy as jnp
from jax import lax
from jax.experimental import pallas as pl
from jax.experimental.pallas import tpu as pltpu
from jax.experimental.pallas import tpu_sc as plsc

N_TOKENS = 100000
D = 128
D4 = 32
B, L = 4096, 200
N = B * L

NC, NS, LANES = 2, 16, 16
NW = NC * NS
NPW = N // NW
K = 128
CHUNKS = NPW // K
RING = 4


_ROWS = 4000


def _tt_body(t0_ref, t1_ref, t2_ref, w_ref, vpw_ref, vpb_ref, pjb_ref,
             tt0_ref, tt1_ref, tt2_ref, uc_ref):
    u = jnp.dot(vpw_ref[...], w_ref[3], preferred_element_type=jnp.float32)
    c = (jnp.dot(vpb_ref[...], w_ref[3], preferred_element_type=jnp.float32)
         + pjb_ref[...])
    tt0_ref[...] = jnp.dot(t0_ref[...], w_ref[0],
                           preferred_element_type=jnp.float32) + c[None, :]
    tt1_ref[...] = jnp.dot(t1_ref[...], w_ref[1],
                           preferred_element_type=jnp.float32)
    tt2_ref[...] = jnp.dot(t2_ref[...], w_ref[2],
                           preferred_element_type=jnp.float32)

    @pl.when(pl.program_id(0) == 0)
    def _():
        uc_ref[...] = jnp.concatenate([u, c[None, :]], axis=0)


def _make_tt(t0, t1, t2, w4, vp_w, vp_b, proj_b):
    grid = N_TOKENS // _ROWS
    tile_spec = pl.BlockSpec((_ROWS, D4), lambda i: (i, 0))
    out_spec = pl.BlockSpec((_ROWS, D), lambda i: (i, 0))
    const = lambda *shape: pl.BlockSpec(shape, lambda i: (0,) * len(shape))
    return pl.pallas_call(
        _tt_body,
        grid=(grid,),
        in_specs=[
            tile_spec, tile_spec, tile_spec,
            const(4, D4, D), const(1, D4), const(D4,), const(D,),
        ],
        out_specs=[out_spec, out_spec, out_spec, const(2, D)],
        out_shape=[
            jax.ShapeDtypeStruct((N_TOKENS, D), jnp.float32),
            jax.ShapeDtypeStruct((N_TOKENS, D), jnp.float32),
            jax.ShapeDtypeStruct((N_TOKENS, D), jnp.float32),
            jax.ShapeDtypeStruct((2, D), jnp.float32),
        ],
    )(t0, t1, t2, w4, vp_w, vp_b, proj_b)


def _sc_body(seqb_hbm, tt0_hbm, tt1_hbm, tt2_hbm, uc_hbm, out_hbm, *scratch):
    it = iter(scratch)
    seqv = [next(it) for _ in range(RING)]
    tiv = [next(it) for _ in range(RING)]
    aiv = [next(it) for _ in range(RING)]
    biv = [next(it) for _ in range(RING)]
    rtv = [next(it) for _ in range(RING)]
    uc_v = next(it)
    gsem = [next(it) for _ in range(RING)]
    osem = [next(it) for _ in range(RING)]

    wid = lax.axis_index("s") * NC + lax.axis_index("c")
    c_base = wid * CHUNKS

    pltpu.sync_copy(uc_hbm, uc_v)
    u_rows = [uc_v[0, pl.ds(16 * j, 16)] for j in range(D // 16)]

    def load_and_fire(ci, b):
        pltpu.sync_copy(seqb_hbm.at[c_base + ci], seqv[b])
        for g in range(K // LANES):
            s = pl.ds(g * LANES, LANES)
            tiv[b][s] = jnp.clip(seqv[b][0, s].astype(jnp.int32),
                                 0, N_TOKENS - 1)
            aiv[b][s] = jnp.clip(seqv[b][1, s].astype(jnp.int32),
                                 0, N_TOKENS - 1)
            biv[b][s] = jnp.clip(seqv[b][2, s].astype(jnp.int32),
                                 0, N_TOKENS - 1)

        def grp_body(g, carry):
            vblk = seqv[b][3, pl.ds(g * LANES, LANES)]
            for t in range(LANES):
                k = g * LANES + t
                vk = jnp.broadcast_to(vblk[t], (LANES,))
                for j in range(D // 16):
                    rtv[b][k, pl.ds(16 * j, 16)] = vk * u_rows[j]
            return carry

        lax.fori_loop(0, K // LANES, grp_body, 0)
        pltpu.async_copy(tt0_hbm.at[tiv[b]], rtv[b], gsem[b], add=True)
        pltpu.async_copy(tt1_hbm.at[aiv[b]], rtv[b], gsem[b], add=True)
        pltpu.async_copy(tt2_hbm.at[biv[b]], rtv[b], gsem[b], add=True)

    def wait_gathers(b):
        pltpu.make_async_copy(tt0_hbm.at[tiv[b]], rtv[b], gsem[b]).wait()
        pltpu.make_async_copy(tt1_hbm.at[aiv[b]], rtv[b], gsem[b]).wait()
        pltpu.make_async_copy(tt2_hbm.at[biv[b]], rtv[b], gsem[b]).wait()

    def out_slice(ci):
        return out_hbm.at[pl.ds((c_base + ci) * K, K)]

    def fire_out(ci, b):
        pltpu.async_copy(rtv[b], out_slice(ci), osem[b])

    def wait_out(ci, b):
        pltpu.make_async_copy(rtv[b], out_slice(ci), osem[b]).wait()

    for b in range(RING - 1):
        load_and_fire(b, b)

    def ring_body(p, carry):
        for q in range(RING):
            n = RING * p + q
            wait_gathers(q)
            fire_out(n, q)
            m = n + RING - 1
            bm = (q + RING - 1) % RING

            @pl.when(m < CHUNKS)
            def _():
                @pl.when(m >= RING)
                def _():
                    wait_out(m - RING, bm)
                load_and_fire(m, bm)
        return carry

    lax.fori_loop(0, CHUNKS // RING, ring_body, 0)
    for q in range(RING):
        wait_out(CHUNKS - RING + q, q)


@functools.cache
def _sc_gather_fn():
    return pl.kernel(
        _sc_body,
        out_type=jax.ShapeDtypeStruct((N, D), jnp.float32),
        mesh=plsc.VectorSubcoreMesh(core_axis_name="c", subcore_axis_name="s",
                                    num_cores=NC, num_subcores=NS),
        compiler_params=pltpu.CompilerParams(needs_layout_passes=False),
        scratch_types=(
            [pltpu.VMEM((4, K), jnp.float32) for _ in range(RING)]
            + [pltpu.VMEM((K,), jnp.int32) for _ in range(3 * RING)]
            + [pltpu.VMEM((K, D), jnp.float32) for _ in range(RING)]
            + [pltpu.VMEM((2, D), jnp.float32)]
            + [pltpu.SemaphoreType.DMA for _ in range(2 * RING)]
        ),
    )


def kernel(seq, type_emb, node_a_emb, node_b_emb, vp_w, vp_b, proj_w, proj_b):
    w4 = proj_w.reshape(4, D4, D)
    tt0, tt1, tt2, uc = _make_tt(type_emb, node_a_emb, node_b_emb,
                                 w4, vp_w, vp_b, proj_b)
    seqb = seq.reshape(N // K, K, 4).transpose(0, 2, 1)
    out = _sc_gather_fn()(seqb, tt0, tt1, tt2, uc)
    return out.reshape(B, L, D)
